# baseline (speedup 1.0000x reference)
import numpy as np

import jax
import jax.numpy as jnp
from jax import lax
from jax.experimental import pallas as pl
from jax.experimental.pallas import tpu as pltpu


_KH = _KW = 5


def _banded_weights(wt, C, OC, OW, KH=_KH, KW=_KW):
    WI = OW + KW - 1
    ow = np.arange(OW)[:, None, None]
    cc = np.arange(C)[None, :, None]
    oc = np.arange(OC)[None, None, :]
    cols = np.broadcast_to(ow * OC + oc, (OW, C, OC)).ravel()
    out = jnp.zeros((KH, WI * C, OW * OC), wt.dtype)
    for i in range(KH):
        for j in range(KW):
            rows = np.broadcast_to((ow + j) * C + cc, (OW, C, OC)).ravel()
            vals = jnp.broadcast_to(wt[i * KW + j][None], (OW, C, OC)).reshape(-1)
            out = out.at[i, rows, cols].set(vals)
    return out


def _banded_weights_per_channel(wt, C, OC, OW, KH=_KH, KW=_KW):
    WI = OW + KW - 1
    ow = np.arange(OW)[:, None]
    oc = np.arange(OC)[None, :]
    cols = (ow * OC + oc).ravel()
    out = jnp.zeros((KH * C, WI, OW * OC), wt.dtype)
    for i in range(KH):
        for j in range(KW):
            rows = np.broadcast_to(ow + j, (OW, OC)).ravel()
            for c in range(C):
                vals = jnp.broadcast_to(wt[i * KW + j, c][None], (OW, OC)).reshape(-1)
                out = out.at[i * C + c, rows, cols].set(vals)
    return out


def _pool_selector(C, PW, off):
    S = np.zeros((2 * PW * C, PW * C), np.float32)
    pw = np.arange(PW)[:, None]
    c = np.arange(C)[None, :]
    S[((2 * pw + off) * C + c).ravel(), (pw * C + c).ravel()] = 1.0
    return jnp.asarray(S)


def _make_body(B):
    M = B * 32
    Mv = M - 4
    M2 = B * 16
    Mv2 = M2 - 4
    M3 = B * 8
    Mv3 = M3 - 4
    f32 = jnp.float32

    def body(x_ref, w1_ref, b1_ref, s1e_ref, s1o_ref,
             w2_ref, b2_ref, s2e_ref, s2o_ref,
             f1_ref, fb1_ref, f2_ref, fb2_ref, f3_ref, fb3_ref, o_ref):
        xf = x_ref[...].reshape(B, 96, 32)
        xcs = [xf[:, 32 * c:32 * (c + 1), :].reshape(M, 32) for c in range(3)]
        a = None
        for i in range(5):
            for c in range(3):
                p = jnp.dot(xcs[c][i:i + Mv], w1_ref[i * 3 + c],
                            preferred_element_type=f32)
                a = p if a is None else a + p
        a = jnp.maximum(a + b1_ref[...], 0.0)
        a = jnp.concatenate([a, jnp.zeros((4, 168), f32)], axis=0)
        cm = jnp.maximum(jnp.dot(a, s1e_ref[...], preferred_element_type=f32),
                         jnp.dot(a, s1o_ref[...], preferred_element_type=f32))
        rm = jnp.max(cm.reshape(M2, 2, 84), axis=1)
        a2 = jnp.dot(rm[0:Mv2], w2_ref[0], preferred_element_type=f32)
        for i in range(1, 5):
            a2 = a2 + jnp.dot(rm[i:i + Mv2], w2_ref[i], preferred_element_type=f32)
        a2 = jnp.maximum(a2 + b2_ref[...], 0.0)
        a2 = jnp.concatenate([a2, jnp.zeros((4, 160), f32)], axis=0)
        cm2 = jnp.maximum(jnp.dot(a2, s2e_ref[...], preferred_element_type=f32),
                          jnp.dot(a2, s2o_ref[...], preferred_element_type=f32))
        rm2 = jnp.max(cm2.reshape(M3, 2, 80), axis=1)
        h = jnp.dot(rm2[0:Mv3], f1_ref[0], preferred_element_type=f32)
        for p in range(1, 5):
            h = h + jnp.dot(rm2[p:p + Mv3], f1_ref[p], preferred_element_type=f32)
        h = jnp.maximum(h + fb1_ref[...], 0.0)
        ri = lax.broadcasted_iota(jnp.int32, (B, Mv3), 0)
        ci = lax.broadcasted_iota(jnp.int32, (B, Mv3), 1)
        sel = (ci == 8 * ri).astype(f32)
        hc = jnp.dot(sel, h, preferred_element_type=f32)
        h2 = jnp.maximum(jnp.dot(hc, f2_ref[...], preferred_element_type=f32)
                         + fb2_ref[...], 0.0)
        o_ref[...] = (jnp.dot(h2, f3_ref[...], preferred_element_type=f32)
                      + fb3_ref[...])

    return body


def kernel(c1_w, c1_b, c2_w, c2_b, fc1_w, fc1_b, fc2_w, fc2_b, fc3_w, fc3_b, x):
    N = x.shape[0]
    B = next(b for b in (128, 64, 32, 16, 8, 4, 2, 1) if N % b == 0)
    M = B * 32

    w1 = _banded_weights_per_channel(c1_w, 3, 6, 28)
    b1 = jnp.tile(c1_b.reshape(1, 6), (1, 28))
    s1e, s1o = _pool_selector(6, 14, 0), _pool_selector(6, 14, 1)
    w2 = _banded_weights(c2_w, 6, 16, 10)
    b2 = jnp.tile(c2_b.reshape(1, 16), (1, 10))
    s2e, s2o = _pool_selector(16, 5, 0), _pool_selector(16, 5, 1)
    f1 = fc1_w.reshape(5, 80, 120)
    fb1 = fc1_b.reshape(1, 120)
    fb2 = fc2_b.reshape(1, 84)
    fb3 = fc3_b.reshape(1, 10)

    x2 = x.reshape(N * 96, 32)

    res = lambda *_: (0, 0)
    resw = lambda *_: (0, 0, 0)
    out = pl.pallas_call(
        _make_body(B),
        out_shape=jax.ShapeDtypeStruct((N, 10), jnp.float32),
        grid=(N // B,),
        in_specs=[
            pl.BlockSpec((B * 96, 32), lambda b: (b, 0)),
            pl.BlockSpec((15, 32, 168), resw),
            pl.BlockSpec((1, 168), res),
            pl.BlockSpec((168, 84), res),
            pl.BlockSpec((168, 84), res),
            pl.BlockSpec((5, 84, 160), resw),
            pl.BlockSpec((1, 160), res),
            pl.BlockSpec((160, 80), res),
            pl.BlockSpec((160, 80), res),
            pl.BlockSpec((5, 80, 120), resw),
            pl.BlockSpec((1, 120), res),
            pl.BlockSpec((120, 84), res),
            pl.BlockSpec((1, 84), res),
            pl.BlockSpec((84, 10), res),
            pl.BlockSpec((1, 10), res),
        ],
        out_specs=pl.BlockSpec((B, 10), lambda b: (b, 0)),
        compiler_params=pltpu.CompilerParams(dimension_semantics=("parallel",)),
    )(x2, w1, b1, s1e, s1o, w2, b2, s2e, s2o, f1, fb1, fc2_w, fb2, fc3_w, fb3)
    return out


# speedup vs baseline: 2.0885x; 2.0885x over previous
import numpy as np

import jax
import jax.numpy as jnp
from jax import lax
from jax.experimental import pallas as pl
from jax.experimental.pallas import tpu as pltpu


_KH = _KW = 5


def _banded_weights(wt, C, OC, OW, KH=_KH, KW=_KW):
    WI = OW + KW - 1
    ow = np.arange(OW)[:, None, None]
    cc = np.arange(C)[None, :, None]
    oc = np.arange(OC)[None, None, :]
    cols = np.broadcast_to(ow * OC + oc, (OW, C, OC)).ravel()
    idx = np.zeros((KH, WI * C, OW * OC), np.int32)
    msk = np.zeros((KH, WI * C, OW * OC), wt.dtype)
    for i in range(KH):
        for j in range(KW):
            rows = np.broadcast_to((ow + j) * C + cc, (OW, C, OC)).ravel()
            src = np.broadcast_to((i * KW + j) * C * OC + cc * OC + oc,
                                  (OW, C, OC)).ravel()
            idx[i, rows, cols] = src
            msk[i, rows, cols] = 1
    return wt.reshape(-1)[jnp.asarray(idx)] * jnp.asarray(msk)


def _banded_weights_per_channel(wt, C, OC, OW, KH=_KH, KW=_KW):
    WI = OW + KW - 1
    ow = np.arange(OW)[:, None]
    oc = np.arange(OC)[None, :]
    cols = (ow * OC + oc).ravel()
    idx = np.zeros((KH * C, WI, OW * OC), np.int32)
    msk = np.zeros((KH * C, WI, OW * OC), wt.dtype)
    for i in range(KH):
        for j in range(KW):
            rows = np.broadcast_to(ow + j, (OW, OC)).ravel()
            for c in range(C):
                src = ((i * KW + j) * C * OC + c * OC + oc)
                idx[i * C + c, rows, cols] = np.broadcast_to(src, (OW, OC)).ravel()
                msk[i * C + c, rows, cols] = 1
    return wt.reshape(-1)[jnp.asarray(idx)] * jnp.asarray(msk)


def _pool_selector(C, PW, off):
    S = np.zeros((2 * PW * C, PW * C), np.float32)
    pw = np.arange(PW)[:, None]
    c = np.arange(C)[None, :]
    S[((2 * pw + off) * C + c).ravel(), (pw * C + c).ravel()] = 1.0
    return jnp.asarray(S)


def _make_body(B):
    M = B * 32
    Mv = M - 4
    M2 = B * 16
    Mv2 = M2 - 4
    M3 = B * 8
    Mv3 = M3 - 4
    f32 = jnp.float32

    def body(x_ref, w1_ref, b1_ref, s1e_ref, s1o_ref,
             w2_ref, b2_ref, s2e_ref, s2o_ref,
             f1_ref, fb1_ref, f2_ref, fb2_ref, f3_ref, fb3_ref, o_ref):
        xf = x_ref[...].reshape(B, 96, 32)
        xcs = [xf[:, 32 * c:32 * (c + 1), :].reshape(M, 32) for c in range(3)]
        a = None
        for i in range(5):
            for c in range(3):
                p = jnp.dot(xcs[c][i:i + Mv], w1_ref[i * 3 + c],
                            preferred_element_type=f32)
                a = p if a is None else a + p
        a = jnp.maximum(a + b1_ref[...], 0.0)
        a = jnp.concatenate([a, jnp.zeros((4, 168), f32)], axis=0)
        cm = jnp.maximum(jnp.dot(a, s1e_ref[...], preferred_element_type=f32),
                         jnp.dot(a, s1o_ref[...], preferred_element_type=f32))
        rm = jnp.max(cm.reshape(M2, 2, 84), axis=1)
        a2 = jnp.dot(rm[0:Mv2], w2_ref[0], preferred_element_type=f32)
        for i in range(1, 5):
            a2 = a2 + jnp.dot(rm[i:i + Mv2], w2_ref[i], preferred_element_type=f32)
        a2 = jnp.maximum(a2 + b2_ref[...], 0.0)
        a2 = jnp.concatenate([a2, jnp.zeros((4, 160), f32)], axis=0)
        cm2 = jnp.maximum(jnp.dot(a2, s2e_ref[...], preferred_element_type=f32),
                          jnp.dot(a2, s2o_ref[...], preferred_element_type=f32))
        rm2 = jnp.max(cm2.reshape(M3, 2, 80), axis=1)
        h = jnp.dot(rm2[0:Mv3], f1_ref[0], preferred_element_type=f32)
        for p in range(1, 5):
            h = h + jnp.dot(rm2[p:p + Mv3], f1_ref[p], preferred_element_type=f32)
        h = jnp.maximum(h + fb1_ref[...], 0.0)
        ri = lax.broadcasted_iota(jnp.int32, (B, Mv3), 0)
        ci = lax.broadcasted_iota(jnp.int32, (B, Mv3), 1)
        sel = (ci == 8 * ri).astype(f32)
        hc = jnp.dot(sel, h, preferred_element_type=f32)
        h2 = jnp.maximum(jnp.dot(hc, f2_ref[...], preferred_element_type=f32)
                         + fb2_ref[...], 0.0)
        o_ref[...] = (jnp.dot(h2, f3_ref[...], preferred_element_type=f32)
                      + fb3_ref[...])

    return body


def kernel(c1_w, c1_b, c2_w, c2_b, fc1_w, fc1_b, fc2_w, fc2_b, fc3_w, fc3_b, x):
    N = x.shape[0]
    B = next(b for b in (128, 64, 32, 16, 8, 4, 2, 1) if N % b == 0)
    M = B * 32

    w1 = _banded_weights_per_channel(c1_w, 3, 6, 28)
    b1 = jnp.tile(c1_b.reshape(1, 6), (1, 28))
    s1e, s1o = _pool_selector(6, 14, 0), _pool_selector(6, 14, 1)
    w2 = _banded_weights(c2_w, 6, 16, 10)
    b2 = jnp.tile(c2_b.reshape(1, 16), (1, 10))
    s2e, s2o = _pool_selector(16, 5, 0), _pool_selector(16, 5, 1)
    f1 = fc1_w.reshape(5, 80, 120)
    fb1 = fc1_b.reshape(1, 120)
    fb2 = fc2_b.reshape(1, 84)
    fb3 = fc3_b.reshape(1, 10)

    x2 = x.reshape(N * 96, 32)

    res = lambda *_: (0, 0)
    resw = lambda *_: (0, 0, 0)
    out = pl.pallas_call(
        _make_body(B),
        out_shape=jax.ShapeDtypeStruct((N, 10), jnp.float32),
        grid=(N // B,),
        in_specs=[
            pl.BlockSpec((B * 96, 32), lambda b: (b, 0)),
            pl.BlockSpec((15, 32, 168), resw),
            pl.BlockSpec((1, 168), res),
            pl.BlockSpec((168, 84), res),
            pl.BlockSpec((168, 84), res),
            pl.BlockSpec((5, 84, 160), resw),
            pl.BlockSpec((1, 160), res),
            pl.BlockSpec((160, 80), res),
            pl.BlockSpec((160, 80), res),
            pl.BlockSpec((5, 80, 120), resw),
            pl.BlockSpec((1, 120), res),
            pl.BlockSpec((120, 84), res),
            pl.BlockSpec((1, 84), res),
            pl.BlockSpec((84, 10), res),
            pl.BlockSpec((1, 10), res),
        ],
        out_specs=pl.BlockSpec((B, 10), lambda b: (b, 0)),
        compiler_params=pltpu.CompilerParams(dimension_semantics=("parallel",)),
    )(x2, w1, b1, s1e, s1o, w2, b2, s2e, s2o, f1, fb1, fc2_w, fb2, fc3_w, fb3)
    return out


# speedup vs baseline: 4.3918x; 2.1029x over previous
import numpy as np

import jax
import jax.numpy as jnp
from jax import lax
from jax.experimental import pallas as pl
from jax.experimental.pallas import tpu as pltpu


_KH = _KW = 5


def _banded_weights(wt, C, OC, OW, KH=_KH, KW=_KW):
    WI = OW + KW - 1
    w_ = np.arange(WI)[None, :, None]
    ow_ = np.arange(OW)[None, None, :]
    j_ = np.arange(KW)[:, None, None]
    sel = jnp.asarray((w_ - ow_ == j_).astype(wt.dtype))
    wr = wt.reshape(KH, KW, C, OC)
    out = jnp.einsum("jwv,ijco->iwcvo", sel, wr)
    return out.reshape(KH, WI * C, OW * OC)


def _banded_weights_per_channel(wt, C, OC, OW, KH=_KH, KW=_KW):
    WI = OW + KW - 1
    w_ = np.arange(WI)[None, :, None]
    ow_ = np.arange(OW)[None, None, :]
    j_ = np.arange(KW)[:, None, None]
    sel = jnp.asarray((w_ - ow_ == j_).astype(wt.dtype))
    wr = wt.reshape(KH, KW, C, OC)
    out = jnp.einsum("jwv,ijco->icwvo", sel, wr)
    return out.reshape(KH * C, WI, OW * OC)


def _pool_selector(C, PW, off):
    S = np.zeros((2 * PW * C, PW * C), np.float32)
    pw = np.arange(PW)[:, None]
    c = np.arange(C)[None, :]
    S[((2 * pw + off) * C + c).ravel(), (pw * C + c).ravel()] = 1.0
    return jnp.asarray(S)


def _make_body(B):
    M = B * 32
    Mv = M - 4
    M2 = B * 16
    Mv2 = M2 - 4
    M3 = B * 8
    Mv3 = M3 - 4
    f32 = jnp.float32

    def body(x_ref, w1_ref, b1_ref, s1e_ref, s1o_ref,
             w2_ref, b2_ref, s2e_ref, s2o_ref,
             f1_ref, fb1_ref, f2_ref, fb2_ref, f3_ref, fb3_ref, o_ref):
        xf = x_ref[...].reshape(B, 96, 32)
        xcs = [xf[:, 32 * c:32 * (c + 1), :].reshape(M, 32) for c in range(3)]
        a = None
        for i in range(5):
            for c in range(3):
                p = jnp.dot(xcs[c][i:i + Mv], w1_ref[i * 3 + c],
                            preferred_element_type=f32)
                a = p if a is None else a + p
        a = jnp.maximum(a + b1_ref[...], 0.0)
        a = jnp.concatenate([a, jnp.zeros((4, 168), f32)], axis=0)
        cm = jnp.maximum(jnp.dot(a, s1e_ref[...], preferred_element_type=f32),
                         jnp.dot(a, s1o_ref[...], preferred_element_type=f32))
        rm = jnp.max(cm.reshape(M2, 2, 84), axis=1)
        a2 = jnp.dot(rm[0:Mv2], w2_ref[0], preferred_element_type=f32)
        for i in range(1, 5):
            a2 = a2 + jnp.dot(rm[i:i + Mv2], w2_ref[i], preferred_element_type=f32)
        a2 = jnp.maximum(a2 + b2_ref[...], 0.0)
        a2 = jnp.concatenate([a2, jnp.zeros((4, 160), f32)], axis=0)
        cm2 = jnp.maximum(jnp.dot(a2, s2e_ref[...], preferred_element_type=f32),
                          jnp.dot(a2, s2o_ref[...], preferred_element_type=f32))
        rm2 = jnp.max(cm2.reshape(M3, 2, 80), axis=1)
        h = jnp.dot(rm2[0:Mv3], f1_ref[0], preferred_element_type=f32)
        for p in range(1, 5):
            h = h + jnp.dot(rm2[p:p + Mv3], f1_ref[p], preferred_element_type=f32)
        h = jnp.maximum(h + fb1_ref[...], 0.0)
        ri = lax.broadcasted_iota(jnp.int32, (B, Mv3), 0)
        ci = lax.broadcasted_iota(jnp.int32, (B, Mv3), 1)
        sel = (ci == 8 * ri).astype(f32)
        hc = jnp.dot(sel, h, preferred_element_type=f32)
        h2 = jnp.maximum(jnp.dot(hc, f2_ref[...], preferred_element_type=f32)
                         + fb2_ref[...], 0.0)
        o_ref[...] = (jnp.dot(h2, f3_ref[...], preferred_element_type=f32)
                      + fb3_ref[...])

    return body


def kernel(c1_w, c1_b, c2_w, c2_b, fc1_w, fc1_b, fc2_w, fc2_b, fc3_w, fc3_b, x):
    N = x.shape[0]
    B = next(b for b in (128, 64, 32, 16, 8, 4, 2, 1) if N % b == 0)
    M = B * 32

    w1 = _banded_weights_per_channel(c1_w, 3, 6, 28)
    b1 = jnp.tile(c1_b.reshape(1, 6), (1, 28))
    s1e, s1o = _pool_selector(6, 14, 0), _pool_selector(6, 14, 1)
    w2 = _banded_weights(c2_w, 6, 16, 10)
    b2 = jnp.tile(c2_b.reshape(1, 16), (1, 10))
    s2e, s2o = _pool_selector(16, 5, 0), _pool_selector(16, 5, 1)
    f1 = fc1_w.reshape(5, 80, 120)
    fb1 = fc1_b.reshape(1, 120)
    fb2 = fc2_b.reshape(1, 84)
    fb3 = fc3_b.reshape(1, 10)

    x2 = x.reshape(N * 96, 32)

    res = lambda *_: (0, 0)
    resw = lambda *_: (0, 0, 0)
    out = pl.pallas_call(
        _make_body(B),
        out_shape=jax.ShapeDtypeStruct((N, 10), jnp.float32),
        grid=(N // B,),
        in_specs=[
            pl.BlockSpec((B * 96, 32), lambda b: (b, 0)),
            pl.BlockSpec((15, 32, 168), resw),
            pl.BlockSpec((1, 168), res),
            pl.BlockSpec((168, 84), res),
            pl.BlockSpec((168, 84), res),
            pl.BlockSpec((5, 84, 160), resw),
            pl.BlockSpec((1, 160), res),
            pl.BlockSpec((160, 80), res),
            pl.BlockSpec((160, 80), res),
            pl.BlockSpec((5, 80, 120), resw),
            pl.BlockSpec((1, 120), res),
            pl.BlockSpec((120, 84), res),
            pl.BlockSpec((1, 84), res),
            pl.BlockSpec((84, 10), res),
            pl.BlockSpec((1, 10), res),
        ],
        out_specs=pl.BlockSpec((B, 10), lambda b: (b, 0)),
        compiler_params=pltpu.CompilerParams(dimension_semantics=("parallel",)),
    )(x2, w1, b1, s1e, s1o, w2, b2, s2e, s2o, f1, fb1, fc2_w, fb2, fc3_w, fb3)
    return out
